# speedup vs baseline: 1.2043x; 1.2043x over previous
import jax
import jax.numpy as jnp
from jax.experimental import pallas as pl
from jax.experimental.pallas import tpu as pltpu

N = 100000
D = 64
EPS = 1e-5
BLK = 2000
C = N // BLK


def _stats_body(x_ref, o_ref, acc_ref):
    i = pl.program_id(0)

    @pl.when(i == 0)
    def _():
        acc_ref[...] = jnp.zeros_like(acc_ref)

    x = x_ref[...]
    acc_ref[0:1, :] += jnp.sum(x, axis=0, keepdims=True)
    acc_ref[1:2, :] += jnp.sum(x * x, axis=0, keepdims=True)

    @pl.when(i == C - 1)
    def _():
        mean = acc_ref[0:1, :] / N
        ex2 = acc_ref[1:2, :] / N
        var = ex2 - mean * mean
        rstd = jax.lax.rsqrt(var + EPS)
        o_ref[...] = jnp.concatenate([mean, rstd], axis=0)


def _norm_body(x_ref, st_ref, o_ref):
    mean = st_ref[0:1, :]
    rstd = st_ref[1:2, :]
    o_ref[...] = (x_ref[...] - mean) * rstd


def kernel(table):
    stats = pl.pallas_call(
        _stats_body,
        grid=(C,),
        in_specs=[pl.BlockSpec((BLK, D), lambda i: (i, 0))],
        out_specs=pl.BlockSpec((2, D), lambda i: (0, 0)),
        out_shape=jax.ShapeDtypeStruct((2, D), jnp.float32),
        scratch_shapes=[pltpu.VMEM((2, D), jnp.float32)],
    )(table)
    normed = pl.pallas_call(
        _norm_body,
        grid=(C,),
        in_specs=[
            pl.BlockSpec((BLK, D), lambda i: (i, 0)),
            pl.BlockSpec((2, D), lambda i: (0, 0)),
        ],
        out_specs=pl.BlockSpec((BLK, D), lambda i: (i, 0)),
        out_shape=jax.ShapeDtypeStruct((N, D), jnp.float32),
    )(table, stats)
    return normed[None, :, None, :]
